# Initial kernel scaffold; baseline (speedup 1.0000x reference)
import jax
import jax.numpy as jnp
from jax.experimental import pallas as pl


def kernel(x, edge_index, W, a_src, a_dst, bias):
    raise NotImplementedError("write your pallas kernel here")


# speedup vs baseline: 41.2512x; 41.2512x over previous
import functools

import jax
import jax.numpy as jnp
from jax import lax
from jax.experimental import pallas as pl
from jax.experimental.pallas import tpu as pltpu
from jax.experimental.pallas import tpu_sc as plsc

N = 10000
E = 320000
IN_F = 128
H = 8
F = 16
HF = H * F

NC = 2
NS = 16
NW = NC * NS
EPW = E // NW
B = 80
NB = EPW // B
RLO = 624
RHI = 640
CK = 16

_mesh = plsc.VectorSubcoreMesh(core_axis_name="c", subcore_axis_name="s")

_GDN = lax.GatherDimensionNumbers(
    offset_dims=(), collapsed_slice_dims=(0,), start_index_map=(0,))


def _lane_gather(v, idx):
    return lax.gather(v, idx[:, None], _GDN, (1,),
                      mode=lax.GatherScatterMode.PROMISE_IN_BOUNDS)


@functools.partial(
    pl.kernel,
    mesh=_mesh,
    compiler_params=pltpu.CompilerParams(use_tc_tiling_on_sc=False),
    out_type=[
        jax.ShapeDtypeStruct((NC, N, HF), jnp.float32),
        jax.ShapeDtypeStruct((NC, N, F), jnp.float32),
    ],
    scratch_types=[
        pltpu.VMEM((B,), jnp.int32),
        pltpu.VMEM((B,), jnp.int32),
        pltpu.VMEM((B,), jnp.int32),
        pltpu.VMEM((B, F), jnp.float32),
        pltpu.VMEM((B, F), jnp.float32),
        pltpu.VMEM((B, HF), jnp.float32),
        pltpu.VMEM((B, HF), jnp.float32),
        pltpu.VMEM((B, F), jnp.float32),
        pltpu.VMEM((CK, HF), jnp.float32),
        pltpu.VMEM((RHI, F), jnp.float32),
        pltpu.VMEM_SHARED((N, HF), jnp.float32),
        pltpu.VMEM_SHARED((N, F), jnp.float32),
        pltpu.SemaphoreType.DMA,
    ],
)
def _edge_sweep(edges_hbm, h_hbm, s_hbm, num_out, den_out,
                ipk, isrc, idst, srows, drows, hrows, msg, wv, tmpr, tmpd,
                num_sh, den_sh, sem):
    c = lax.axis_index("c")
    s = lax.axis_index("s")
    wid = s * NC + c

    zero16 = jnp.zeros((16,), jnp.float32)

    def _zr(i, carry):
        for j in range(HF // 16):
            tmpr[i, pl.ds(16 * j, 16)] = zero16
        return carry
    lax.fori_loop(0, CK, _zr, 0)

    def _zd(i, carry):
        tmpd[i, :] = zero16
        return carry
    lax.fori_loop(0, RHI, _zd, 0)

    last = s == NS - 1
    nrows = jnp.where(last, RHI, RLO)
    nck = nrows // CK
    r0 = s * RLO

    def _z2(i, carry):
        pltpu.sync_copy(tmpr, num_sh.at[pl.ds(r0 + i * CK, CK)])
        return carry
    lax.fori_loop(0, nck, _z2, 0)

    @pl.when(jnp.logical_not(last))
    def _():
        pltpu.sync_copy(tmpd.at[pl.ds(0, RLO)], den_sh.at[pl.ds(r0, RLO)])

    @pl.when(last)
    def _():
        pltpu.sync_copy(tmpd, den_sh.at[pl.ds(r0, RHI)])

    plsc.subcore_barrier()

    lane = lax.iota(jnp.int32, 16)
    hi_idx = (lane & 7) + 8
    lo_mask = lane < 8
    e0 = wid * EPW

    def _batch(bi, carry):
        base = e0 + bi * B
        pltpu.sync_copy(edges_hbm.at[pl.ds(base, B)], ipk)
        for j in range(B // 16):
            v = ipk[pl.ds(16 * j, 16)]
            isrc[pl.ds(16 * j, 16)] = v & 0xFFFF
            idst[pl.ds(16 * j, 16)] = v >> 16
        pltpu.async_copy(s_hbm.at[isrc], srows, sem).wait()
        pltpu.async_copy(s_hbm.at[idst], drows, sem).wait()
        pltpu.async_copy(h_hbm.at[isrc], hrows, sem).wait()

        def _edge(b, ecarry):
            srow = srows[b, :]
            drow = drows[b, :]
            dpart = _lane_gather(drow, hi_idx)
            a = srow + dpart
            a = jnp.where(a > 0, a, 0.2 * a)
            w16 = jnp.where(lo_mask, jnp.exp(a), 0.0)
            wv[b, :] = w16
            for h in range(H):
                bc = _lane_gather(w16, jnp.full((16,), h, jnp.int32))
                msg[b, pl.ds(16 * h, 16)] = bc * hrows[b, pl.ds(16 * h, 16)]
            return ecarry
        lax.fori_loop(0, B, _edge, 0)

        pltpu.sync_copy(wv, den_sh.at[idst], add=True)
        pltpu.sync_copy(msg, num_sh.at[idst], add=True)
        return carry
    lax.fori_loop(0, NB, _batch, 0)

    plsc.subcore_barrier()

    def _rb(i, carry):
        pltpu.sync_copy(num_sh.at[pl.ds(r0 + i * CK, CK)], tmpr)
        pltpu.sync_copy(tmpr, num_out.at[c, pl.ds(r0 + i * CK, CK)])
        return carry
    lax.fori_loop(0, nck, _rb, 0)

    @pl.when(jnp.logical_not(last))
    def _():
        pltpu.sync_copy(den_sh.at[pl.ds(r0, RLO)], tmpd.at[pl.ds(0, RLO)])
        pltpu.sync_copy(tmpd.at[pl.ds(0, RLO)], den_out.at[c, pl.ds(r0, RLO)])

    @pl.when(last)
    def _():
        pltpu.sync_copy(den_sh.at[pl.ds(r0, RHI)], tmpd)
        pltpu.sync_copy(tmpd, den_out.at[c, pl.ds(r0, RHI)])


def _prep_body(x_ref, wt_ref, a_ref, h_ref, s_ref):
    hb = jnp.dot(x_ref[...], wt_ref[...], preferred_element_type=jnp.float32)
    h_ref[...] = hb
    s_ref[...] = jnp.dot(hb, a_ref[...], preferred_element_type=jnp.float32)


_BN = 1000


def _prep(x, wt, a_mat):
    return pl.pallas_call(
        _prep_body,
        grid=(N // _BN,),
        in_specs=[
            pl.BlockSpec((_BN, IN_F), lambda i: (i, 0)),
            pl.BlockSpec((IN_F, HF), lambda i: (0, 0)),
            pl.BlockSpec((HF, F), lambda i: (0, 0)),
        ],
        out_specs=[
            pl.BlockSpec((_BN, HF), lambda i: (i, 0)),
            pl.BlockSpec((_BN, F), lambda i: (i, 0)),
        ],
        out_shape=[
            jax.ShapeDtypeStruct((N, HF), jnp.float32),
            jax.ShapeDtypeStruct((N, F), jnp.float32),
        ],
    )(x, wt, a_mat)


def _fin_body(n_ref, d_ref, b_ref, r_ref, o_ref):
    ns = n_ref[0] + n_ref[1]
    den = d_ref[0] + d_ref[1]
    rec = 1.0 / jnp.clip(den, 1e-10, None)
    recf = jnp.dot(rec, r_ref[...], preferred_element_type=jnp.float32)
    o_ref[...] = ns * recf + b_ref[...]


def _fin(num, den, bias2d, r_mat):
    return pl.pallas_call(
        _fin_body,
        grid=(N // _BN,),
        in_specs=[
            pl.BlockSpec((NC, _BN, HF), lambda i: (0, i, 0)),
            pl.BlockSpec((NC, _BN, F), lambda i: (0, i, 0)),
            pl.BlockSpec((1, HF), lambda i: (0, 0)),
            pl.BlockSpec((F, HF), lambda i: (0, 0)),
        ],
        out_specs=pl.BlockSpec((_BN, HF), lambda i: (i, 0)),
        out_shape=jax.ShapeDtypeStruct((N, HF), jnp.float32),
    )(num, den, bias2d, r_mat)


def kernel(x, edge_index, W, a_src, a_dst, bias):
    eye8 = jnp.eye(H, dtype=jnp.float32)
    a1 = (a_src[:, :, None] * eye8[:, None, :]).reshape(HF, H)
    a2 = (a_dst[:, :, None] * eye8[:, None, :]).reshape(HF, H)
    a_mat = jnp.concatenate([a1, a2], axis=1)

    h, s = _prep(x, W.T, a_mat)

    packed = edge_index[0] | (edge_index[1] << 16)
    num, den = _edge_sweep(packed, h, s)

    col_head = jnp.arange(HF, dtype=jnp.int32) // F
    rows = jnp.arange(F, dtype=jnp.int32)
    r_mat = ((rows[:, None] == col_head[None, :]) & (rows[:, None] < H)
             ).astype(jnp.float32)
    return _fin(num, den, bias.reshape(1, HF), r_mat)
